# baseline (speedup 1.0000x reference)
import jax
import jax.numpy as jnp
from jax.experimental import pallas as pl
from jax.experimental.pallas import tpu as pltpu

N = 16384
IN_DIM = 512
H1 = 256
H2 = 128
OUT_DIM = 32

BLOCK = 2048
G = N // (2 * BLOCK)


def _mlp(x_ref, w1, b1, w2, b2, w3, b3):
    h = jnp.dot(x_ref[...].astype(jnp.bfloat16), w1,
                preferred_element_type=jnp.float32)
    h = jnp.maximum(h + b1, 0.0)
    h = jnp.dot(h.astype(jnp.bfloat16), w2, preferred_element_type=jnp.float32)
    h = jnp.maximum(h + b2, 0.0)
    h = jnp.dot(h.astype(jnp.bfloat16), w3, preferred_element_type=jnp.float32)
    return h + b3


def _mlp_block(w1_hbm, b1_hbm, w2_hbm, b2_hbm, w3_hbm, b3_hbm, xa_ref, xb_ref,
               out_ref, w1_s, b1_s, w2_s, b2_s, w3_s, b3_s, sems):
    i = pl.program_id(0)

    @pl.when(i == 0)
    def _():
        pltpu.make_async_copy(w1_hbm, w1_s, sems.at[0]).start()
        pltpu.make_async_copy(b1_hbm, b1_s, sems.at[1]).start()
        pltpu.make_async_copy(w2_hbm, w2_s, sems.at[2]).start()
        pltpu.make_async_copy(b2_hbm, b2_s, sems.at[3]).start()
        pltpu.make_async_copy(w3_hbm, w3_s, sems.at[4]).start()
        pltpu.make_async_copy(b3_hbm, b3_s, sems.at[5]).start()
        pltpu.make_async_copy(w1_hbm, w1_s, sems.at[0]).wait()
        pltpu.make_async_copy(b1_hbm, b1_s, sems.at[1]).wait()
        pltpu.make_async_copy(w2_hbm, w2_s, sems.at[2]).wait()
        pltpu.make_async_copy(b2_hbm, b2_s, sems.at[3]).wait()
        pltpu.make_async_copy(w3_hbm, w3_s, sems.at[4]).wait()
        pltpu.make_async_copy(b3_hbm, b3_s, sems.at[5]).wait()

    w1 = w1_s[...].astype(jnp.bfloat16)
    w2 = w2_s[...].astype(jnp.bfloat16)
    w3 = w3_s[...].astype(jnp.bfloat16)
    b1 = b1_s[...]
    b2 = b2_s[...]
    b3 = b3_s[...]
    out_ref[:BLOCK] = _mlp(xa_ref, w1, b1, w2, b2, w3, b3)
    out_ref[BLOCK:] = _mlp(xb_ref, w1, b1, w2, b2, w3, b3)


def kernel(x, W1, b1, W2, b2, W3, b3):
    b1r = b1.reshape(1, H1)
    b2r = b2.reshape(1, H2)
    b3r = b3.reshape(1, OUT_DIM)
    hbm = pl.BlockSpec(memory_space=pltpu.MemorySpace.HBM)
    return pl.pallas_call(
        _mlp_block,
        grid=(G,),
        in_specs=[
            hbm, hbm, hbm, hbm, hbm, hbm,
            pl.BlockSpec((BLOCK, IN_DIM), lambda i: (2 * i, 0)),
            pl.BlockSpec((BLOCK, IN_DIM), lambda i: (2 * i + 1, 0)),
        ],
        out_specs=pl.BlockSpec((2 * BLOCK, OUT_DIM), lambda i: (i, 0)),
        out_shape=jax.ShapeDtypeStruct((N, OUT_DIM), jnp.float32),
        scratch_shapes=[
            pltpu.VMEM((IN_DIM, H1), jnp.float32),
            pltpu.VMEM((1, H1), jnp.float32),
            pltpu.VMEM((H1, H2), jnp.float32),
            pltpu.VMEM((1, H2), jnp.float32),
            pltpu.VMEM((H2, OUT_DIM), jnp.float32),
            pltpu.VMEM((1, OUT_DIM), jnp.float32),
            pltpu.SemaphoreType.DMA((6,)),
        ],
        compiler_params=pltpu.CompilerParams(
            dimension_semantics=("arbitrary",),
        ),
    )(W1, b1r, W2, b2r, W3, b3r, x, x)


# speedup vs baseline: 1.0083x; 1.0083x over previous
import jax
import jax.numpy as jnp
from jax.experimental import pallas as pl
from jax.experimental.pallas import tpu as pltpu

N = 16384
IN_DIM = 512
H1 = 256
H2 = 128
OUT_DIM = 32
OUT_PAD = 128

BLOCK = 2048
G = N // (2 * BLOCK)


def _mlp(x_ref, w1, b1, w2, b2, w3, b3):
    h = jnp.dot(x_ref[...].astype(jnp.bfloat16), w1,
                preferred_element_type=jnp.float32)
    h = jnp.maximum(h + b1, 0.0)
    h = jnp.dot(h.astype(jnp.bfloat16), w2, preferred_element_type=jnp.float32)
    h = jnp.maximum(h + b2, 0.0)
    h = jnp.dot(h.astype(jnp.bfloat16), w3, preferred_element_type=jnp.float32)
    return h + b3


def _mlp_block(xa_ref, xb_ref, w1_ref, b1_ref, w2_ref, b2_ref, w3_ref, b3_ref,
               out_ref):
    w1 = w1_ref[...].astype(jnp.bfloat16)
    w2 = w2_ref[...].astype(jnp.bfloat16)
    w3 = w3_ref[...].astype(jnp.bfloat16)
    b1 = b1_ref[...]
    b2 = b2_ref[...]
    b3 = b3_ref[...]
    out_ref[:BLOCK] = _mlp(xa_ref, w1, b1, w2, b2, w3, b3)
    out_ref[BLOCK:] = _mlp(xb_ref, w1, b1, w2, b2, w3, b3)


def kernel(x, W1, b1, W2, b2, W3, b3):
    b1r = b1.reshape(1, H1)
    b2r = b2.reshape(1, H2)
    W3p = jnp.pad(W3, ((0, 0), (0, OUT_PAD - OUT_DIM)))
    b3p = jnp.pad(b3, (0, OUT_PAD - OUT_DIM)).reshape(1, OUT_PAD)
    out = pl.pallas_call(
        _mlp_block,
        grid=(G,),
        in_specs=[
            pl.BlockSpec((BLOCK, IN_DIM), lambda i: (2 * i, 0)),
            pl.BlockSpec((BLOCK, IN_DIM), lambda i: (2 * i + 1, 0)),
            pl.BlockSpec((IN_DIM, H1), lambda i: (0, 0)),
            pl.BlockSpec((1, H1), lambda i: (0, 0)),
            pl.BlockSpec((H1, H2), lambda i: (0, 0)),
            pl.BlockSpec((1, H2), lambda i: (0, 0)),
            pl.BlockSpec((H2, OUT_PAD), lambda i: (0, 0)),
            pl.BlockSpec((1, OUT_PAD), lambda i: (0, 0)),
        ],
        out_specs=pl.BlockSpec((2 * BLOCK, OUT_PAD), lambda i: (i, 0)),
        out_shape=jax.ShapeDtypeStruct((N, OUT_PAD), jnp.float32),
        compiler_params=pltpu.CompilerParams(
            dimension_semantics=("arbitrary",),
        ),
    )(x, x, W1, b1r, W2, b2r, W3p, b3p)
    return out[:, :OUT_DIM]
